# baseline (speedup 1.0000x reference)
import functools

import jax
import jax.numpy as jnp
from jax import lax
from jax.experimental import pallas as pl
from jax.experimental.pallas import tpu as pltpu
from jax.experimental.pallas import tpu_sc as plsc

B = 16384
NSPLIT = 2
BS = B // NSPLIT
D = 128
H1, H2, H3 = 64, 32, 16
L = 16
NC = 2
NS = 16
NW = NC * NS
BPW = BS // NW
CHUNK = 128
NCHUNK = BPW // CHUNK
NK = D // L


def _sc_gather_body(uids, iids, tug, tig, tum, tim, wpg,
                    ogp, oum, oim,
                    idx_u, idx_i, a0, a1, b0, b1, c0, c1, p0, wg_v,
                    sa0, sa1, sb0, sb1, sc0, sc1, spw0, spw1, swc0, swc1, six):
    wid = lax.axis_index("s") * NC + lax.axis_index("c")
    base = wid * BPW
    ix_cps = []
    for c in range(NCHUNK):
        ix_cps.append(pltpu.async_copy(
            uids.at[pl.ds(base + c * CHUNK, CHUNK)], idx_u.at[c], six))
        ix_cps.append(pltpu.async_copy(
            iids.at[pl.ds(base + c * CHUNK, CHUNK)], idx_i.at[c], six))
    pltpu.sync_copy(wpg, wg_v)
    for cp in ix_cps:
        cp.wait()
    wg = [wg_v[pl.ds(k * L, L)] for k in range(NK)]

    abufs = (a0, a1)
    bbufs = (b0, b1)
    cbufs = (c0, c1)
    sa = (sa0, sa1)
    sb = (sb0, sb1)
    sc = (sc0, sc1)
    spw = (spw0, spw1)
    swc = (swc0, swc1)

    ga = [None, None]
    gb = [None, None]
    gc = [None, None]
    pw = [None]
    wc = [None, None]
    for g in (0, 1):
        ga[g] = pltpu.async_copy(tug.at[idx_u.at[g]], abufs[g], sa[g])
        gb[g] = pltpu.async_copy(tig.at[idx_i.at[g]], bbufs[g], sb[g])

    mlp_tasks = ([(tum, oum, idx_u, c) for c in range(NCHUNK)]
                 + [(tim, oim, idx_i, c) for c in range(NCHUNK)])
    prev = None
    for k, (tbl, out, idx, c) in enumerate(mlp_tasks):
        slot = k % 2
        if wc[slot] is not None:
            wc[slot].wait()
        gc[slot] = pltpu.async_copy(tbl.at[idx.at[c]], cbufs[slot], sc[slot])
        if prev is not None:
            ps, pout, prow = prev
            gc[ps].wait()
            wc[ps] = pltpu.async_copy(cbufs[ps], pout.at[pl.ds(prow, CHUNK)],
                                      swc[ps])
        prev = (slot, out, base + c * CHUNK)
        if k % 2 == 1:
            g = k // 2
            gs = g % 2
            ga[gs].wait()
            gb[gs].wait()
            if pw[0] is not None:
                pw[0].wait()
            a_ref, b_ref, p_ref = abufs[gs], bbufs[gs], p0

            def row_body(i, _, a_ref=a_ref, b_ref=b_ref, p_ref=p_ref):
                acc = a_ref[i, pl.ds(0, L)] * b_ref[i, pl.ds(0, L)] * wg[0]
                for kk in range(1, NK):
                    acc = acc + (a_ref[i, pl.ds(kk * L, L)]
                                 * b_ref[i, pl.ds(kk * L, L)] * wg[kk])
                p_ref[i, :] = acc
                return 0

            lax.fori_loop(0, CHUNK, row_body, 0, unroll=4)
            pw[0] = pltpu.async_copy(
                p_ref, ogp.at[pl.ds(base + g * CHUNK, CHUNK)], spw[0])
            if g + 2 < NCHUNK:
                ga[gs] = pltpu.async_copy(tug.at[idx_u.at[g + 2]], abufs[gs],
                                          sa[gs])
                gb[gs] = pltpu.async_copy(tig.at[idx_i.at[g + 2]], bbufs[gs],
                                          sb[gs])
    ps, pout, prow = prev
    gc[ps].wait()
    wc[ps] = pltpu.async_copy(cbufs[ps], pout.at[pl.ds(prow, CHUNK)], swc[ps])
    for s in (0, 1):
        if wc[s] is not None:
            wc[s].wait()
    if pw[0] is not None:
        pw[0].wait()


@functools.cache
def _sc_gather():
    return pl.kernel(
        _sc_gather_body,
        out_type=[
            jax.ShapeDtypeStruct((BS, L), jnp.float32),
            jax.ShapeDtypeStruct((BS, D), jnp.float32),
            jax.ShapeDtypeStruct((BS, D), jnp.float32),
        ],
        mesh=plsc.VectorSubcoreMesh(core_axis_name="c", subcore_axis_name="s",
                                    num_cores=NC, num_subcores=NS),
        scratch_types=[
            pltpu.VMEM((NCHUNK, CHUNK), jnp.int32),
            pltpu.VMEM((NCHUNK, CHUNK), jnp.int32),
            pltpu.VMEM((CHUNK, D), jnp.float32),
            pltpu.VMEM((CHUNK, D), jnp.float32),
            pltpu.VMEM((CHUNK, D), jnp.float32),
            pltpu.VMEM((CHUNK, D), jnp.float32),
            pltpu.VMEM((CHUNK, D), jnp.float32),
            pltpu.VMEM((CHUNK, D), jnp.float32),
            pltpu.VMEM((CHUNK, L), jnp.float32),
            pltpu.VMEM((D,), jnp.float32),
        ] + [pltpu.SemaphoreType.DMA] * 11,
    )


_CONTRACT_00 = (((0,), (0,)), ((), ()))
_CONTRACT_01 = (((0,), (1,)), ((), ()))


def _tc_body(gp, um, im, w1u, w1i, b1, w2, b2, w3, b3, wph, bp, out):
    h = lax.dot_general(w1u[...], um[...], _CONTRACT_01,
                        preferred_element_type=jnp.float32)
    h = h + lax.dot_general(w1i[...], im[...], _CONTRACT_01,
                            preferred_element_type=jnp.float32)
    h = jnp.maximum(h + b1[...], 0.0)
    h = jnp.maximum(
        lax.dot_general(w2[...], h, _CONTRACT_00,
                        preferred_element_type=jnp.float32) + b2[...], 0.0)
    h = jnp.maximum(
        lax.dot_general(w3[...], h, _CONTRACT_00,
                        preferred_element_type=jnp.float32) + b3[...], 0.0)
    s = lax.dot_general(wph[...], h, _CONTRACT_00,
                        preferred_element_type=jnp.float32)
    ones = jnp.ones((L, 1), jnp.float32)
    s = s + lax.dot_general(ones, gp[...], _CONTRACT_01,
                            preferred_element_type=jnp.float32)
    out[...] = jax.nn.sigmoid(s + bp[...])[None]


RBLK = 4096


def _tc_call(gp, um, im, w1u, w1i, b1c, w2, b2c, w3, b3c, wphc, bpc):
    rb = lambda i: (i, 0)
    z = lambda i: (0, 0)
    return pl.pallas_call(
        _tc_body,
        grid=(BS // RBLK,),
        in_specs=[
            pl.BlockSpec((RBLK, L), rb),
            pl.BlockSpec((RBLK, D), rb),
            pl.BlockSpec((RBLK, D), rb),
            pl.BlockSpec((D, H1), z),
            pl.BlockSpec((D, H1), z),
            pl.BlockSpec((H1, 1), z),
            pl.BlockSpec((H1, H2), z),
            pl.BlockSpec((H2, 1), z),
            pl.BlockSpec((H2, H3), z),
            pl.BlockSpec((H3, 1), z),
            pl.BlockSpec((H3, 1), z),
            pl.BlockSpec((1, 1), z),
        ],
        out_specs=pl.BlockSpec((1, 1, RBLK), lambda i: (i, 0, 0)),
        out_shape=jax.ShapeDtypeStruct((BS // RBLK, 1, RBLK), jnp.float32),
    )(gp, um, im, w1u, w1i, b1c, w2, b2c, w3, b3c, wphc, bpc)


def kernel(user_ids, item_ids, user_emb_gmf, item_emb_gmf, user_emb_mlp,
           item_emb_mlp, W1, b1, W2, b2, W3, b3, Wp, bp):
    uids = user_ids.astype(jnp.int32)
    iids = item_ids.astype(jnp.int32)
    sc = _sc_gather()
    parts = []
    for sl in range(NSPLIT):
        lo, hi = sl * BS, (sl + 1) * BS
        parts.append(sc(uids[lo:hi], iids[lo:hi], user_emb_gmf, item_emb_gmf,
                        user_emb_mlp, item_emb_mlp, Wp[:D, 0]))
    outs = []
    for gp, um, im in parts:
        outs.append(_tc_call(gp, um, im, W1[:D], W1[D:], b1.reshape(H1, 1),
                             W2, b2.reshape(H2, 1), W3, b3.reshape(H3, 1),
                             Wp[D:, 0].reshape(H3, 1),
                             bp.reshape(1, 1)).reshape(BS))
    return jnp.concatenate(outs)


# speedup vs baseline: 1.0744x; 1.0744x over previous
import functools

import jax
import jax.numpy as jnp
from jax import lax
from jax.experimental import pallas as pl
from jax.experimental.pallas import tpu as pltpu
from jax.experimental.pallas import tpu_sc as plsc

B = 16384
D = 128
H1, H2, H3 = 64, 32, 16
L = 16
NC = 2
NS = 16
NW = NC * NS
BPW = B // NW
CHUNK = 128
NCHUNK = BPW // CHUNK
NK = D // L


def _sc_gather_body(uids, iids, tug, tig, tum, tim, wpg,
                    ogp, oum, oim,
                    idx_u, idx_i, a0, a1, b0, b1, c0, c1, p0, wg_v,
                    sa0, sa1, sb0, sb1, sc0, sc1, spw0, spw1, swc0, swc1, six):
    wid = lax.axis_index("s") * NC + lax.axis_index("c")
    base = wid * BPW
    ix_cps = []
    for c in range(NCHUNK):
        ix_cps.append(pltpu.async_copy(
            uids.at[pl.ds(base + c * CHUNK, CHUNK)], idx_u.at[c], six))
        ix_cps.append(pltpu.async_copy(
            iids.at[pl.ds(base + c * CHUNK, CHUNK)], idx_i.at[c], six))
    pltpu.sync_copy(wpg, wg_v)
    for cp in ix_cps:
        cp.wait()
    wg = [wg_v[pl.ds(k * L, L)] for k in range(NK)]

    abufs = (a0, a1)
    bbufs = (b0, b1)
    cbufs = (c0, c1)
    sa = (sa0, sa1)
    sb = (sb0, sb1)
    sc = (sc0, sc1)
    spw = (spw0, spw1)
    swc = (swc0, swc1)

    ga = [None, None]
    gb = [None, None]
    gc = [None, None]
    pw = [None]
    wc = [None, None]
    for g in (0, 1):
        ga[g] = pltpu.async_copy(tug.at[idx_u.at[g]], abufs[g], sa[g])
        gb[g] = pltpu.async_copy(tig.at[idx_i.at[g]], bbufs[g], sb[g])

    mlp_tasks = ([(tum, oum, idx_u, c) for c in range(NCHUNK)]
                 + [(tim, oim, idx_i, c) for c in range(NCHUNK)])
    prev = None
    for k, (tbl, out, idx, c) in enumerate(mlp_tasks):
        slot = k % 2
        if wc[slot] is not None:
            wc[slot].wait()
        gc[slot] = pltpu.async_copy(tbl.at[idx.at[c]], cbufs[slot], sc[slot])
        if prev is not None:
            ps, pout, prow = prev
            gc[ps].wait()
            wc[ps] = pltpu.async_copy(cbufs[ps], pout.at[pl.ds(prow, CHUNK)],
                                      swc[ps])
        prev = (slot, out, base + c * CHUNK)
        if k % 2 == 1:
            g = k // 2
            gs = g % 2
            ga[gs].wait()
            gb[gs].wait()
            if pw[0] is not None:
                pw[0].wait()
            a_ref, b_ref, p_ref = abufs[gs], bbufs[gs], p0

            def row_body(i, _, a_ref=a_ref, b_ref=b_ref, p_ref=p_ref):
                acc = a_ref[i, pl.ds(0, L)] * b_ref[i, pl.ds(0, L)] * wg[0]
                for kk in range(1, NK):
                    acc = acc + (a_ref[i, pl.ds(kk * L, L)]
                                 * b_ref[i, pl.ds(kk * L, L)] * wg[kk])
                p_ref[i, :] = acc
                return 0

            lax.fori_loop(0, CHUNK, row_body, 0, unroll=4)
            pw[0] = pltpu.async_copy(
                p_ref, ogp.at[pl.ds(base + g * CHUNK, CHUNK)], spw[0])
            if g + 2 < NCHUNK:
                ga[gs] = pltpu.async_copy(tug.at[idx_u.at[g + 2]], abufs[gs],
                                          sa[gs])
                gb[gs] = pltpu.async_copy(tig.at[idx_i.at[g + 2]], bbufs[gs],
                                          sb[gs])
    ps, pout, prow = prev
    gc[ps].wait()
    wc[ps] = pltpu.async_copy(cbufs[ps], pout.at[pl.ds(prow, CHUNK)], swc[ps])
    for s in (0, 1):
        if wc[s] is not None:
            wc[s].wait()
    if pw[0] is not None:
        pw[0].wait()


@functools.cache
def _sc_gather():
    return pl.kernel(
        _sc_gather_body,
        out_type=[
            jax.ShapeDtypeStruct((B, L), jnp.float32),
            jax.ShapeDtypeStruct((B, D), jnp.float32),
            jax.ShapeDtypeStruct((B, D), jnp.float32),
        ],
        mesh=plsc.VectorSubcoreMesh(core_axis_name="c", subcore_axis_name="s",
                                    num_cores=NC, num_subcores=NS),
        scratch_types=[
            pltpu.VMEM((NCHUNK, CHUNK), jnp.int32),
            pltpu.VMEM((NCHUNK, CHUNK), jnp.int32),
            pltpu.VMEM((CHUNK, D), jnp.float32),
            pltpu.VMEM((CHUNK, D), jnp.float32),
            pltpu.VMEM((CHUNK, D), jnp.float32),
            pltpu.VMEM((CHUNK, D), jnp.float32),
            pltpu.VMEM((CHUNK, D), jnp.float32),
            pltpu.VMEM((CHUNK, D), jnp.float32),
            pltpu.VMEM((CHUNK, L), jnp.float32),
            pltpu.VMEM((D,), jnp.float32),
        ] + [pltpu.SemaphoreType.DMA] * 11,
    )


_CONTRACT_00 = (((0,), (0,)), ((), ()))
_CONTRACT_01 = (((0,), (1,)), ((), ()))


def _tc_body(gp, um, im, w1u, w1i, b1, w2, b2, w3, b3, wph, bp, out):
    h = lax.dot_general(w1u[...], um[...], _CONTRACT_01,
                        preferred_element_type=jnp.float32)
    h = h + lax.dot_general(w1i[...], im[...], _CONTRACT_01,
                            preferred_element_type=jnp.float32)
    h = jnp.maximum(h + b1[...], 0.0)
    h = jnp.maximum(
        lax.dot_general(w2[...], h, _CONTRACT_00,
                        preferred_element_type=jnp.float32) + b2[...], 0.0)
    h = jnp.maximum(
        lax.dot_general(w3[...], h, _CONTRACT_00,
                        preferred_element_type=jnp.float32) + b3[...], 0.0)
    s = lax.dot_general(wph[...], h, _CONTRACT_00,
                        preferred_element_type=jnp.float32)
    ones = jnp.ones((L, 1), jnp.float32)
    s = s + lax.dot_general(ones, gp[...], _CONTRACT_01,
                            preferred_element_type=jnp.float32)
    out[...] = jax.nn.sigmoid(s + bp[...])[None]


RBLK = 4096


def _tc_call(gp, um, im, w1u, w1i, b1c, w2, b2c, w3, b3c, wphc, bpc):
    rb = lambda i: (i, 0)
    z = lambda i: (0, 0)
    return pl.pallas_call(
        _tc_body,
        grid=(B // RBLK,),
        in_specs=[
            pl.BlockSpec((RBLK, L), rb),
            pl.BlockSpec((RBLK, D), rb),
            pl.BlockSpec((RBLK, D), rb),
            pl.BlockSpec((D, H1), z),
            pl.BlockSpec((D, H1), z),
            pl.BlockSpec((H1, 1), z),
            pl.BlockSpec((H1, H2), z),
            pl.BlockSpec((H2, 1), z),
            pl.BlockSpec((H2, H3), z),
            pl.BlockSpec((H3, 1), z),
            pl.BlockSpec((H3, 1), z),
            pl.BlockSpec((1, 1), z),
        ],
        out_specs=pl.BlockSpec((1, 1, RBLK), lambda i: (i, 0, 0)),
        out_shape=jax.ShapeDtypeStruct((B // RBLK, 1, RBLK), jnp.float32),
    )(gp, um, im, w1u, w1i, b1c, w2, b2c, w3, b3c, wphc, bpc)


def kernel(user_ids, item_ids, user_emb_gmf, item_emb_gmf, user_emb_mlp,
           item_emb_mlp, W1, b1, W2, b2, W3, b3, Wp, bp):
    uids = user_ids.astype(jnp.int32)
    iids = item_ids.astype(jnp.int32)
    gp, um, im = _sc_gather()(uids, iids, user_emb_gmf, item_emb_gmf,
                              user_emb_mlp, item_emb_mlp, Wp[:D, 0])
    out = _tc_call(gp, um, im, W1[:D], W1[D:], b1.reshape(H1, 1), W2,
                   b2.reshape(H2, 1), W3, b3.reshape(H3, 1),
                   Wp[D:, 0].reshape(H3, 1), bp.reshape(1, 1))
    return out.reshape(B)


# speedup vs baseline: 1.0873x; 1.0120x over previous
import functools

import jax
import jax.numpy as jnp
from jax import lax
from jax.experimental import pallas as pl
from jax.experimental.pallas import tpu as pltpu
from jax.experimental.pallas import tpu_sc as plsc

B = 16384
D = 128
H1, H2, H3 = 64, 32, 16
L = 16
NC = 2
NS = 16
NW = NC * NS
BPW = B // NW
CHUNK = 128
NCHUNK = BPW // CHUNK
NK = D // L


def _sc_gather_body(uids, iids, tug, tig, tum, tim, wpg,
                    ogp, oum, oim,
                    idx_u, idx_i, a0, a1, b0, b1, c0, c1, p0, wg_v,
                    sa0, sa1, sb0, sb1, sc0, sc1, spw0, spw1, swc0, swc1, six):
    wid = lax.axis_index("s") * NC + lax.axis_index("c")
    base = wid * BPW
    early, late = [], []
    for c in range(2):
        early.append(pltpu.async_copy(
            uids.at[pl.ds(base + c * CHUNK, CHUNK)], idx_u.at[c], six))
        early.append(pltpu.async_copy(
            iids.at[pl.ds(base + c * CHUNK, CHUNK)], idx_i.at[c], six))
    late.append(pltpu.async_copy(wpg, wg_v, six))
    for c in range(2, NCHUNK):
        late.append(pltpu.async_copy(
            uids.at[pl.ds(base + c * CHUNK, CHUNK)], idx_u.at[c], six))
        late.append(pltpu.async_copy(
            iids.at[pl.ds(base + c * CHUNK, CHUNK)], idx_i.at[c], six))
    for cp in early:
        cp.wait()

    abufs = (a0, a1)
    bbufs = (b0, b1)
    cbufs = (c0, c1)
    sa = (sa0, sa1)
    sb = (sb0, sb1)
    sc = (sc0, sc1)
    spw = (spw0, spw1)
    swc = (swc0, swc1)

    ga = [None, None]
    gb = [None, None]
    gc = [None, None]
    pw = [None]
    wc = [None, None]
    for g in (0, 1):
        ga[g] = pltpu.async_copy(tug.at[idx_u.at[g]], abufs[g], sa[g])
        gb[g] = pltpu.async_copy(tig.at[idx_i.at[g]], bbufs[g], sb[g])
    for cp in late:
        cp.wait()
    wg = [wg_v[pl.ds(k * L, L)] for k in range(NK)]

    mlp_tasks = ([(tum, oum, idx_u, c) for c in range(NCHUNK)]
                 + [(tim, oim, idx_i, c) for c in range(NCHUNK)])
    prev = None
    for k, (tbl, out, idx, c) in enumerate(mlp_tasks):
        slot = k % 2
        if wc[slot] is not None:
            wc[slot].wait()
        gc[slot] = pltpu.async_copy(tbl.at[idx.at[c]], cbufs[slot], sc[slot])
        if prev is not None:
            ps, pout, prow = prev
            gc[ps].wait()
            wc[ps] = pltpu.async_copy(cbufs[ps], pout.at[pl.ds(prow, CHUNK)],
                                      swc[ps])
        prev = (slot, out, base + c * CHUNK)
        if k % 2 == 1:
            g = k // 2
            gs = g % 2
            ga[gs].wait()
            gb[gs].wait()
            if pw[0] is not None:
                pw[0].wait()
            a_ref, b_ref, p_ref = abufs[gs], bbufs[gs], p0

            def row_body(i, _, a_ref=a_ref, b_ref=b_ref, p_ref=p_ref):
                acc = a_ref[i, pl.ds(0, L)] * b_ref[i, pl.ds(0, L)] * wg[0]
                for kk in range(1, NK):
                    acc = acc + (a_ref[i, pl.ds(kk * L, L)]
                                 * b_ref[i, pl.ds(kk * L, L)] * wg[kk])
                p_ref[i, :] = acc
                return 0

            lax.fori_loop(0, CHUNK, row_body, 0, unroll=4)
            pw[0] = pltpu.async_copy(
                p_ref, ogp.at[pl.ds(base + g * CHUNK, CHUNK)], spw[0])
            if g + 2 < NCHUNK:
                ga[gs] = pltpu.async_copy(tug.at[idx_u.at[g + 2]], abufs[gs],
                                          sa[gs])
                gb[gs] = pltpu.async_copy(tig.at[idx_i.at[g + 2]], bbufs[gs],
                                          sb[gs])
    ps, pout, prow = prev
    gc[ps].wait()
    wc[ps] = pltpu.async_copy(cbufs[ps], pout.at[pl.ds(prow, CHUNK)], swc[ps])
    for s in (0, 1):
        if wc[s] is not None:
            wc[s].wait()
    if pw[0] is not None:
        pw[0].wait()


@functools.cache
def _sc_gather():
    return pl.kernel(
        _sc_gather_body,
        out_type=[
            jax.ShapeDtypeStruct((B, L), jnp.float32),
            jax.ShapeDtypeStruct((B, D), jnp.float32),
            jax.ShapeDtypeStruct((B, D), jnp.float32),
        ],
        mesh=plsc.VectorSubcoreMesh(core_axis_name="c", subcore_axis_name="s",
                                    num_cores=NC, num_subcores=NS),
        scratch_types=[
            pltpu.VMEM((NCHUNK, CHUNK), jnp.int32),
            pltpu.VMEM((NCHUNK, CHUNK), jnp.int32),
            pltpu.VMEM((CHUNK, D), jnp.float32),
            pltpu.VMEM((CHUNK, D), jnp.float32),
            pltpu.VMEM((CHUNK, D), jnp.float32),
            pltpu.VMEM((CHUNK, D), jnp.float32),
            pltpu.VMEM((CHUNK, D), jnp.float32),
            pltpu.VMEM((CHUNK, D), jnp.float32),
            pltpu.VMEM((CHUNK, L), jnp.float32),
            pltpu.VMEM((D,), jnp.float32),
        ] + [pltpu.SemaphoreType.DMA] * 11,
    )


_CONTRACT_00 = (((0,), (0,)), ((), ()))
_CONTRACT_01 = (((0,), (1,)), ((), ()))


def _tc_body(gp, um, im, w1u, w1i, b1, w2, b2, w3, b3, wph, bp, out):
    h = lax.dot_general(w1u[...], um[...], _CONTRACT_01,
                        preferred_element_type=jnp.float32)
    h = h + lax.dot_general(w1i[...], im[...], _CONTRACT_01,
                            preferred_element_type=jnp.float32)
    h = jnp.maximum(h + b1[...], 0.0)
    h = jnp.maximum(
        lax.dot_general(w2[...], h, _CONTRACT_00,
                        preferred_element_type=jnp.float32) + b2[...], 0.0)
    h = jnp.maximum(
        lax.dot_general(w3[...], h, _CONTRACT_00,
                        preferred_element_type=jnp.float32) + b3[...], 0.0)
    s = lax.dot_general(wph[...], h, _CONTRACT_00,
                        preferred_element_type=jnp.float32)
    ones = jnp.ones((L, 1), jnp.float32)
    s = s + lax.dot_general(ones, gp[...], _CONTRACT_01,
                            preferred_element_type=jnp.float32)
    out[...] = jax.nn.sigmoid(s + bp[...])[None]


RBLK = 4096


def _tc_call(gp, um, im, w1u, w1i, b1c, w2, b2c, w3, b3c, wphc, bpc):
    rb = lambda i: (i, 0)
    z = lambda i: (0, 0)
    return pl.pallas_call(
        _tc_body,
        grid=(B // RBLK,),
        in_specs=[
            pl.BlockSpec((RBLK, L), rb),
            pl.BlockSpec((RBLK, D), rb),
            pl.BlockSpec((RBLK, D), rb),
            pl.BlockSpec((D, H1), z),
            pl.BlockSpec((D, H1), z),
            pl.BlockSpec((H1, 1), z),
            pl.BlockSpec((H1, H2), z),
            pl.BlockSpec((H2, 1), z),
            pl.BlockSpec((H2, H3), z),
            pl.BlockSpec((H3, 1), z),
            pl.BlockSpec((H3, 1), z),
            pl.BlockSpec((1, 1), z),
        ],
        out_specs=pl.BlockSpec((1, 1, RBLK), lambda i: (i, 0, 0)),
        out_shape=jax.ShapeDtypeStruct((B // RBLK, 1, RBLK), jnp.float32),
    )(gp, um, im, w1u, w1i, b1c, w2, b2c, w3, b3c, wphc, bpc)


def kernel(user_ids, item_ids, user_emb_gmf, item_emb_gmf, user_emb_mlp,
           item_emb_mlp, W1, b1, W2, b2, W3, b3, Wp, bp):
    uids = user_ids.astype(jnp.int32)
    iids = item_ids.astype(jnp.int32)
    gp, um, im = _sc_gather()(uids, iids, user_emb_gmf, item_emb_gmf,
                              user_emb_mlp, item_emb_mlp, Wp[:D, 0])
    out = _tc_call(gp, um, im, W1[:D], W1[D:], b1.reshape(H1, 1), W2,
                   b2.reshape(H2, 1), W3, b3.reshape(H3, 1),
                   Wp[D:, 0].reshape(H3, 1), bp.reshape(1, 1))
    return out.reshape(B)
